# baseline (speedup 1.0000x reference)
import functools

import jax
import jax.numpy as jnp
from jax import lax
from jax.experimental import pallas as pl
from jax.experimental.pallas import tpu as pltpu
from jax.experimental.pallas import tpu_sc as plsc

VOCAB = 1000000
DIM = 32
B = 4096
L = 200

NUM_WORKERS = 32
CHUNK = 128
N_FLAT = B * L
N_CHUNKS = N_FLAT // CHUNK
CHUNKS_PER_WORKER = N_CHUNKS // NUM_WORKERS
BT = B // CHUNK
CD = CHUNK * DIM
QROW = 4 * DIM
K = 2
NT = CHUNKS_PER_WORKER // (2 * K)


def _make_sc_gather():
  mesh = plsc.VectorSubcoreMesh(core_axis_name="c", subcore_axis_name="s")

  @functools.partial(
      pl.kernel,
      mesh=mesh,
      out_type=jax.ShapeDtypeStruct((L, DIM // 8, BT, 8, CHUNK), jnp.float32),
      compiler_params=pltpu.CompilerParams(
          use_tc_tiling_on_sc=True, needs_layout_passes=False),
      scratch_types=[
          pltpu.VMEM((CHUNKS_PER_WORKER, CHUNK), jnp.int32),
          pltpu.VMEM((2 * K, CHUNK), jnp.int32),
          pltpu.VMEM((2 * K, CHUNK, QROW), jnp.float32),
          pltpu.VMEM((2 * K, DIM, CHUNK), jnp.float32),
          pltpu.SemaphoreType.DMA,
          pltpu.SemaphoreType.DMA,
          pltpu.SemaphoreType.DMA,
          pltpu.SemaphoreType.DMA,
      ],
  )
  def gather_kernel(table_hbm, idx_hbm, out_hbm, idx_v, qidx_v, rows_v, ct_v,
                    ga, gb, sa, sb):
    wid = lax.axis_index("s") * 2 + lax.axis_index("c")
    chunk_base = wid * CHUNKS_PER_WORKER
    pltpu.sync_copy(idx_hbm.at[pl.ds(chunk_base, CHUNKS_PER_WORKER)], idx_v)

    iota16 = lax.iota(jnp.int32, 16)
    row_vecs = [iota16 + (i0 * 16) for i0 in range(8)]

    def fire_gather(t, half, s, sem):
      slot = half * K + s
      for i0 in range(8):
        q = idx_v[t, pl.ds(i0 * 16, 16)] >> 2
        qidx_v[slot, pl.ds(i0 * 16, 16)] = q
      pltpu.make_async_copy(
          table_hbm.at[qidx_v.at[slot]], rows_v.at[slot], ga if sem is None
          else sem).start()

    def wait_gather(half, s, sem):
      slot = half * K + s
      pltpu.make_async_copy(
          table_hbm.at[qidx_v.at[slot]], rows_v.at[slot], sem).wait()

    def stores(t, half, s, sem):
      j = chunk_base + t
      lrow = j // BT
      btile = j % BT
      slot = half * K + s
      return [
          pltpu.make_async_copy(
              ct_v.at[slot, pl.ds(dt * 8, 8)],
              out_hbm.at[lrow, dt, btile], sem)
          for dt in range(4)
      ]

    def transpose_half(half, tbase):
      def sbody(s, carry):
        t = tbase + s
        slot = half * K + s
        src = rows_v.at[slot]
        dst = ct_v.at[slot]
        mvecs = [(idx_v[t, pl.ds(i0 * 16, 16)] & 3) << 5 for i0 in range(8)]
        for d in range(DIM):
          vs = [
              plsc.load_gather(src, [row_vecs[i0], mvecs[i0] + d])
              for i0 in range(8)
          ]
          for i0, v in enumerate(vs):
            dst[d, pl.ds(i0 * 16, 16)] = v
        return carry

      lax.fori_loop(0, K, sbody, 0)

    for s in range(K):
      fire_gather(s, 0, s, ga)

    def body(t, carry):
      base = t * 2 * K
      for s in range(K):
        wait_gather(0, s, ga)
      @pl.when(t > 0)
      def _():
        for s in range(K):
          for c in stores(base - K + s, 1, s, sb):
            c.wait()
      for s in range(K):
        fire_gather(base + K + s, 1, s, gb)
      transpose_half(0, base)
      for s in range(K):
        for c in stores(base + s, 0, s, sa):
          c.start()
      for s in range(K):
        wait_gather(1, s, gb)
      transpose_half(1, base + K)
      @pl.when(t < NT - 1)
      def _():
        for s in range(K):
          fire_gather(base + 2 * K + s, 0, s, ga)
      for s in range(K):
        for c in stores(base + s, 0, s, sa):
          c.wait()
      for s in range(K):
        for c in stores(base + K + s, 1, s, sb):
          c.start()
      return carry

    lax.fori_loop(0, NT, body, 0)
    last = (NT - 1) * 2 * K + K
    for s in range(K):
      for c in stores(last + s, 1, s, sb):
        c.wait()

  return gather_kernel


_sc_gather = _make_sc_gather()


def kernel(inp, table):
  idx = jnp.swapaxes(inp, 0, 1).astype(jnp.int32).reshape(N_CHUNKS, CHUNK)
  table4 = table.reshape(VOCAB // 4, QROW)
  out5 = _sc_gather(table4, idx)
  return out5.transpose(2, 4, 0, 1, 3).reshape(B, L, DIM)


# speedup vs baseline: 1.1386x; 1.1386x over previous
import functools

import jax
import jax.numpy as jnp
from jax import lax
from jax.experimental import pallas as pl
from jax.experimental.pallas import tpu as pltpu
from jax.experimental.pallas import tpu_sc as plsc

VOCAB = 1000000
DIM = 32
B = 4096
L = 200

NUM_WORKERS = 32
CHUNK = 128
N_FLAT = B * L
PER_WORKER = N_FLAT // NUM_WORKERS
CHUNKS_PER_WORKER = PER_WORKER // CHUNK


K = 10
NT = CHUNKS_PER_WORKER // (2 * K)


def _make_sc_gather():
  mesh = plsc.VectorSubcoreMesh(core_axis_name="c", subcore_axis_name="s")

  @functools.partial(
      pl.kernel,
      mesh=mesh,
      out_type=jax.ShapeDtypeStruct((N_FLAT, DIM), jnp.float32),
      compiler_params=pltpu.CompilerParams(use_tc_tiling_on_sc=False),
      scratch_types=[
          pltpu.VMEM((CHUNKS_PER_WORKER, CHUNK), jnp.int32),
          pltpu.VMEM((2, K, CHUNK, DIM), jnp.float32),
          pltpu.SemaphoreType.DMA,
          pltpu.SemaphoreType.DMA,
          pltpu.SemaphoreType.DMA,
          pltpu.SemaphoreType.DMA,
      ],
  )
  def gather_kernel(table_hbm, idx_hbm, out_hbm, idx_v, rows_v, ga, gb, sa, sb):
    wid = lax.axis_index("s") * 2 + lax.axis_index("c")
    chunk_base = wid * CHUNKS_PER_WORKER
    pltpu.sync_copy(idx_hbm.at[pl.ds(chunk_base, CHUNKS_PER_WORKER)], idx_v)

    def gath(j, buf_half, b, sem):
      return pltpu.make_async_copy(
          table_hbm.at[idx_v.at[j]], rows_v.at[buf_half, b], sem)

    def store(j, buf_half, b, sem):
      row0 = (chunk_base + j) * CHUNK
      return pltpu.make_async_copy(
          rows_v.at[buf_half, b], out_hbm.at[pl.ds(row0, CHUNK)], sem)

    for b in range(K):
      gath(b, 0, b, ga).start()

    def body(t, carry):
      base = t * 2 * K
      for b in range(K):
        gath(base + b, 0, b, ga).wait()
      @pl.when(t > 0)
      def _():
        for b in range(K):
          store(base - K + b, 1, b, sb).wait()
      for b in range(K):
        gath(base + K + b, 1, b, gb).start()
      for b in range(K):
        store(base + b, 0, b, sa).start()
      for b in range(K):
        gath(base + K + b, 1, b, gb).wait()
      for b in range(K):
        store(base + b, 0, b, sa).wait()
      @pl.when(t < NT - 1)
      def _():
        for b in range(K):
          gath(base + 2 * K + b, 0, b, ga).start()
      for b in range(K):
        store(base + K + b, 1, b, sb).start()
      return carry

    lax.fori_loop(0, NT, body, 0)
    last = (NT - 1) * 2 * K + K
    for b in range(K):
      store(last + b, 1, b, sb).wait()

  return gather_kernel


_sc_gather = _make_sc_gather()


def kernel(inp, table):
  idx = inp.astype(jnp.int32).reshape(N_FLAT // CHUNK, CHUNK)
  out = _sc_gather(table, idx)
  return out.reshape(B, L, DIM)
